# baseline (speedup 1.0000x reference)
import functools

import jax
import jax.numpy as jnp
from jax import lax
from jax.experimental import pallas as pl
from jax.experimental.pallas import tpu as pltpu
from jax.experimental.pallas import tpu_sc as plsc

N = 10000
F16 = 16
DEPTH = 6
LEAD = 4

NW = 32
CHUNK = 128
NCH = 80
EPAD = NW * NCH * CHUNK
ACC_N = 10240
SROWS = ACC_N // 16
BN = 1000

_mesh = plsc.VectorSubcoreMesh(core_axis_name="c", subcore_axis_name="s")
_sc_params = pltpu.CompilerParams(use_tc_tiling_on_sc=False)


PW = NCH * CHUNK
PE = 320000 // NW
PT = 320000 // 16
NCHF = PE // CHUNK
TAIL = PE - NCHF * CHUNK
_NITER = ((NCHF + LEAD + DEPTH - 1) // DEPTH) * DEPTH


@functools.partial(
    pl.kernel,
    out_type=[pltpu.HBM((2, 4, ACC_N, F16), jnp.float32),
              pltpu.HBM((4, ACC_N, F16), jnp.float32)],
    mesh=_mesh,
    compiler_params=pltpu.CompilerParams(use_tc_tiling_on_sc=False,
                                         needs_layout_passes=False),
    scratch_types=[
        [pltpu.VMEM_SHARED((ACC_N, F16), jnp.float32) for _ in range(4)],
        pltpu.VMEM_SHARED((8, ACC_N), jnp.float32),
        pltpu.VMEM_SHARED((ACC_N, F16), jnp.float32),
        pltpu.VMEM((PE,), jnp.int32),
        pltpu.VMEM((PE,), jnp.int32),
        pltpu.VMEM((ACC_N,), jnp.float32),
        pltpu.VMEM((16 * SROWS,), jnp.float32),
        pltpu.VMEM((4 * SROWS,), jnp.float32),
        pltpu.VMEM((SROWS, F16), jnp.float32),
        [pltpu.VMEM((CHUNK, F16), jnp.float32) for _ in range(DEPTH)],
        pltpu.VMEM((TAIL, F16), jnp.float32),
        [pltpu.SemaphoreType.DMA for _ in range(DEPTH)],
        [pltpu.SemaphoreType.DMA for _ in range(DEPTH)],
        pltpu.SemaphoreType.DMA,
    ],
)
def _mega4(h0, h1, h2, h3, e0, e1, e2, e3, zc, macc, dout, accs, part_sh,
           g_sh, sidx_v, didx_v, cnt_v, tmp_v, dinvc_v, hbuf2, bufs,
           tbuf, gsems, ssems, tsem):
    cid = lax.axis_index("c")
    sid = lax.axis_index("s")
    wid = sid * 2 + cid
    row0 = sid * SROWS

    hs = (h0, h1, h2, h3)
    es = (e0, e1, e2, e3)

    z16 = jnp.zeros((16,), jnp.float32)
    one16 = jnp.ones((16,), jnp.float32)

    for acc in accs:
        pltpu.sync_copy(zc, acc.at[pl.ds(row0, SROWS)])

    for k, eref in enumerate(es):
        @pl.loop(0, ACC_N // 16)
        def _(i):
            cnt_v[pl.ds(i * 16, 16)] = z16

        for half in range(2):
            pltpu.sync_copy(eref.at[1, pl.ds(sid * PT + half * PE, PE)],
                            didx_v)

            @pl.loop(0, PE // 64)
            def _(j):
                for l in range(4):
                    idx16 = didx_v[pl.ds(j * 64 + l * 16, 16)]
                    plsc.addupdate_scatter(cnt_v, [idx16], one16)

            for t16 in range((PE % 64) // 16):
                idx16 = didx_v[pl.ds((PE // 64) * 64 + t16 * 16, 16)]
                plsc.addupdate_scatter(cnt_v, [idx16], one16)

        for r in range(2):
            @pl.when((sid >= r * 8) & (sid < (r + 1) * 8))
            def _():
                pltpu.sync_copy(cnt_v, part_sh.at[sid % 8])

            plsc.subcore_barrier()
            for t in range(8):
                pltpu.sync_copy(part_sh.at[t, pl.ds(row0, SROWS)],
                                tmp_v.at[pl.ds((r * 8 + t) * SROWS, SROWS)])
            plsc.subcore_barrier()

        @pl.loop(0, SROWS // 16)
        def _(q):
            acc16 = tmp_v[pl.ds(q * 16, 16)]
            for t in range(1, 16):
                acc16 = acc16 + tmp_v[pl.ds(t * SROWS + q * 16, 16)]
            d = 1.0 + acc16
            i32 = plsc.bitcast(d, jnp.int32)
            i32 = jnp.int32(0x5F3759DF) - (i32 >> 1)
            y = plsc.bitcast(i32, jnp.float32)
            for _ in range(3):
                y = y * (1.5 - 0.5 * d * y * y)
            dinvc_v[pl.ds(k * SROWS + q * 16, 16)] = y
            rows16 = q * 16 + lax.iota(jnp.int32, 16)
            for fcol in range(16):
                plsc.store_scatter(
                    hbuf2, [rows16, jnp.full((16,), fcol, jnp.int32)], y)

        @pl.when(cid == (k // 2))
        def _():
            pltpu.sync_copy(hbuf2, dout.at[k, pl.ds(row0, SROWS)])

    for k, (eref, acc) in enumerate(zip(es, accs)):
        @pl.when(sid < 15)
        def _():
            pltpu.sync_copy(hs[k].at[pl.ds(sid * 640, 640)], hbuf2)

        @pl.when(sid == 15)
        def _():
            pltpu.sync_copy(hs[k].at[pl.ds(9600, N - 9600)],
                            hbuf2.at[pl.ds(0, N - 9600)])

        @pl.loop(0, SROWS // 16)
        def _(q):
            rows16 = q * 16 + lax.iota(jnp.int32, 16)
            dinv16 = dinvc_v[pl.ds(k * SROWS + q * 16, 16)]
            for fcol in range(16):
                cvec = jnp.full((16,), fcol, jnp.int32)
                hcol = plsc.load_gather(hbuf2, [rows16, cvec])
                plsc.store_scatter(hbuf2, [rows16, cvec], hcol * dinv16)

        pltpu.sync_copy(hbuf2, g_sh.at[pl.ds(sid * 640, 640)])

        pltpu.sync_copy(eref.at[0, pl.ds(wid * PE, PE)], sidx_v)
        pltpu.sync_copy(eref.at[1, pl.ds(wid * PE, PE)], didx_v)
        plsc.subcore_barrier()

        def s_at(c):
            return sidx_v.at[pl.ds(pl.multiple_of(c * CHUNK, CHUNK), CHUNK)]

        def d_at(c):
            return didx_v.at[pl.ds(pl.multiple_of(c * CHUNK, CHUNK), CHUNK)]

        @pl.loop(0, _NITER, step=DEPTH)
        def _(j):
            for b in range(DEPTH):
                jj = j + b

                @pl.when(jj < NCHF)
                def _():
                    @pl.when(jj >= DEPTH)
                    def _():
                        pltpu.make_async_copy(
                            bufs[b], acc.at[d_at(jj - DEPTH)],
                            ssems[b]).wait()

                    pltpu.async_copy(g_sh.at[s_at(jj)], bufs[b], gsems[b])

                jk = jj - LEAD
                b2 = (b - LEAD) % DEPTH

                @pl.when((jk >= 0) & (jk < NCHF))
                def _():
                    pltpu.make_async_copy(
                        g_sh.at[s_at(jk)], bufs[b2], gsems[b2]).wait()
                    pltpu.async_copy(bufs[b2], acc.at[d_at(jk)],
                                     ssems[b2], add=True)

        for b in range(DEPTH):
            c = NCHF - DEPTH + b
            pltpu.make_async_copy(
                bufs[c % DEPTH], acc.at[d_at(c)], ssems[c % DEPTH]).wait()

        tidx_s = sidx_v.at[pl.ds(NCHF * CHUNK, TAIL)]
        tidx_d = didx_v.at[pl.ds(NCHF * CHUNK, TAIL)]
        pltpu.async_copy(g_sh.at[tidx_s], tbuf, tsem).wait()
        pltpu.async_copy(tbuf, acc.at[tidx_d], tsem, add=True).wait()

        plsc.subcore_barrier()

    for k, acc in enumerate(accs):
        pltpu.sync_copy(acc.at[pl.ds(row0, SROWS)],
                        macc.at[cid, k, pl.ds(row0, SROWS)])


def _make_msg_kernel(nsets, width):
    @functools.partial(
        pl.kernel,
        out_type=jax.ShapeDtypeStruct((2, nsets, ACC_N, width), jnp.float32),
        mesh=_mesh,
        compiler_params=_sc_params,
        scratch_types=[
            [pltpu.VMEM_SHARED((ACC_N, width), jnp.float32)
             for _ in range(nsets)],
            pltpu.VMEM((PE,), jnp.int32),
            pltpu.VMEM((PE,), jnp.int32),
            [pltpu.VMEM((CHUNK, width), jnp.float32) for _ in range(DEPTH)],
            pltpu.VMEM((TAIL, width), jnp.float32),
            pltpu.VMEM((SROWS, width), jnp.float32),
            pltpu.VMEM_SHARED((ACC_N, width), jnp.float32),
            [pltpu.SemaphoreType.DMA for _ in range(DEPTH)],
            [pltpu.SemaphoreType.DMA for _ in range(DEPTH)],
            pltpu.SemaphoreType.DMA,
        ],
    )
    def msg_kernel(*args):
        gs = args[:nsets]
        erefs = args[nsets:2 * nsets]
        out = args[2 * nsets]
        (accs, sidx_v, didx_v, bufs, tbuf, zeros_v, g_sh, gsems,
         ssems, tsem) = args[2 * nsets + 1:]

        cid = lax.axis_index("c")
        sid = lax.axis_index("s")
        wid = sid * 2 + cid
        row0 = sid * SROWS

        for w0 in range(0, width, 16):
            z = jnp.zeros((16,), jnp.float32)

            @pl.loop(0, SROWS)
            def _(i):
                zeros_v[i, pl.ds(w0, 16)] = z

        for acc in accs:
            pltpu.sync_copy(zeros_v, acc.at[pl.ds(row0, SROWS)])
        plsc.subcore_barrier()

        for g, eref, acc in zip(gs, erefs, accs):
            @pl.when(sid < 15)
            def _():
                pltpu.sync_copy(g.at[pl.ds(sid * 640, 640)],
                                g_sh.at[pl.ds(sid * 640, 640)])

            @pl.when(sid == 15)
            def _():
                pltpu.sync_copy(g.at[pl.ds(9600, N - 9600)],
                                g_sh.at[pl.ds(9600, N - 9600)])

            pltpu.sync_copy(eref.at[0, pl.ds(wid * PE, PE)], sidx_v)
            pltpu.sync_copy(eref.at[1, pl.ds(wid * PE, PE)], didx_v)
            plsc.subcore_barrier()

            def s_at(c):
                return sidx_v.at[pl.ds(pl.multiple_of(c * CHUNK, CHUNK),
                                       CHUNK)]

            def d_at(c):
                return didx_v.at[pl.ds(pl.multiple_of(c * CHUNK, CHUNK),
                                       CHUNK)]

            @pl.loop(0, _NITER, step=DEPTH)
            def _(j):
                for b in range(DEPTH):
                    jj = j + b

                    @pl.when(jj < NCHF)
                    def _():
                        @pl.when(jj >= DEPTH)
                        def _():
                            pltpu.make_async_copy(
                                bufs[b], acc.at[d_at(jj - DEPTH)],
                                ssems[b]).wait()

                        pltpu.async_copy(g_sh.at[s_at(jj)], bufs[b],
                                         gsems[b])

                    jk = jj - LEAD
                    b2 = (b - LEAD) % DEPTH

                    @pl.when((jk >= 0) & (jk < NCHF))
                    def _():
                        pltpu.make_async_copy(
                            g_sh.at[s_at(jk)], bufs[b2], gsems[b2]).wait()
                        pltpu.async_copy(bufs[b2], acc.at[d_at(jk)],
                                         ssems[b2], add=True)

            for b in range(DEPTH):
                c = NCHF - DEPTH + b
                pltpu.make_async_copy(
                    bufs[c % DEPTH], acc.at[d_at(c)], ssems[c % DEPTH]).wait()

            tidx_s = sidx_v.at[pl.ds(NCHF * CHUNK, TAIL)]
            tidx_d = didx_v.at[pl.ds(NCHF * CHUNK, TAIL)]
            pltpu.async_copy(g_sh.at[tidx_s], tbuf, tsem).wait()
            pltpu.async_copy(tbuf, acc.at[tidx_d], tsem, add=True).wait()

            plsc.subcore_barrier()
        for k, acc in enumerate(accs):
            pltpu.sync_copy(acc.at[pl.ds(row0, SROWS)],
                            out.at[cid, k, pl.ds(row0, SROWS)])

    return msg_kernel


_msg1 = _make_msg_kernel(1, F16)


def _tc1a_body(x0_ref, x1_ref, x2_ref, x3_ref, w1_ref, w2_ref, w3_ref,
               h0_ref, h1_ref, h2_ref, h3_ref):
    outs = (h0_ref, h1_ref, h2_ref, h3_ref)
    xs = (x0_ref, x1_ref, x2_ref, x3_ref)
    ws = (w1_ref, w2_ref, w3_ref, w2_ref)
    for k in range(4):
        outs[k][...] = jnp.dot(xs[k][...], ws[k][...],
                               preferred_element_type=jnp.float32)


def _tc1b_body(h0_ref, h1_ref, h2_ref, h3_ref, dinv_ref,
               g0_ref, g1_ref, g2_ref, g3_ref):
    outs = (g0_ref, g1_ref, g2_ref, g3_ref)
    hs = (h0_ref, h1_ref, h2_ref, h3_ref)
    for k in range(4):
        outs[k][...] = hs[k][...] * dinv_ref[k]


def _tc2_body(h0_ref, h1_ref, h2_ref, h3_ref, macc_ref, dinv_ref, bs_ref,
              u_ref):
    hr = (h0_ref, h1_ref, h2_ref, h3_ref)
    outs = []
    for k in range(4):
        acc = macc_ref[0, k] + macc_ref[1, k]
        outs.append(dinv_ref[k] * (acc + hr[k][...] * dinv_ref[k])
                    + bs_ref[k])
    xp = outs[0]
    x = jnp.maximum(xp, 0.0)
    xc1 = outs[1] + x + xp
    x = jnp.maximum(xc1, 0.0)
    xc2 = outs[2] + x + xc1
    x = jnp.maximum(xc2, 0.0)
    xf = outs[3] + x + xc1 + xc2
    x4 = jnp.maximum(xf, 0.0)
    u_ref[...] = x4 * dinv_ref[3]


def _tc3_body(u_ref, m5_ref, dinv_ref, we_ref, be_ref, o_ref):
    dinv = dinv_ref[0][:, 0:1]
    s = m5_ref[0, 0] + m5_ref[1, 0] + u_ref[...]
    h = jnp.dot(s, we_ref[...], preferred_element_type=jnp.float32)
    logits = dinv * h + be_ref[...]
    mx = jnp.max(logits, axis=1, keepdims=True)
    lse = mx + jnp.log(jnp.sum(jnp.exp(logits - mx), axis=1, keepdims=True))
    o_ref[...] = logits - lse


def kernel(x_parent, x_child1, x_child2, x_final_descendants,
           edge_index_parent, edge_index_child1, edge_index_child2,
           edge_index_final_descendants,
           W1, b1, W2, b2, W3, b3, We, be):
    f = jnp.float32
    bs = jnp.stack([b1, b2, b3, b2])

    F0, F1, F2 = W1.shape[0], W2.shape[0], W3.shape[0]
    h0, h1, h2, h3 = pl.pallas_call(
        _tc1a_body,
        grid=(N // BN,),
        in_specs=[
            pl.BlockSpec((BN, F0), lambda i: (i, 0)),
            pl.BlockSpec((BN, F1), lambda i: (i, 0)),
            pl.BlockSpec((BN, F2), lambda i: (i, 0)),
            pl.BlockSpec((BN, F1), lambda i: (i, 0)),
            pl.BlockSpec((F0, F16), lambda i: (0, 0)),
            pl.BlockSpec((F1, F16), lambda i: (0, 0)),
            pl.BlockSpec((F2, F16), lambda i: (0, 0)),
        ],
        out_specs=[pl.BlockSpec((BN, F16), lambda i: (i, 0))] * 4,
        out_shape=[jax.ShapeDtypeStruct((N, F16), f)] * 4,
    )(x_parent, x_child1, x_child2, x_final_descendants, W1, W2, W3)

    zc = jnp.zeros((SROWS, F16), f)
    macc, dinvb = _mega4(h0, h1, h2, h3, edge_index_parent,
                         edge_index_child1, edge_index_child2,
                         edge_index_final_descendants, zc)

    u = pl.pallas_call(
        _tc2_body,
        grid=(N // BN,),
        in_specs=[pl.BlockSpec((BN, F16), lambda i: (i, 0))] * 4 + [
            pl.BlockSpec((2, 4, BN, F16), lambda i: (0, 0, i, 0)),
            pl.BlockSpec((4, BN, F16), lambda i: (0, i, 0)),
            pl.BlockSpec((4, F16), lambda i: (0, 0)),
        ],
        out_specs=pl.BlockSpec((BN, F16), lambda i: (i, 0)),
        out_shape=jax.ShapeDtypeStruct((N, F16), f),
    )(h0, h1, h2, h3, macc, dinvb, bs)

    m5 = _msg1(u, edge_index_final_descendants)

    out = pl.pallas_call(
        _tc3_body,
        grid=(N // BN,),
        in_specs=[
            pl.BlockSpec((BN, F16), lambda i: (i, 0)),
            pl.BlockSpec((2, 1, BN, F16), lambda i: (0, 0, i, 0)),
            pl.BlockSpec((1, BN, F16), lambda i: (3, i, 0)),
            pl.BlockSpec((F16, 40), lambda i: (0, 0)),
            pl.BlockSpec((40,), lambda i: (0,)),
        ],
        out_specs=pl.BlockSpec((BN, 40), lambda i: (i, 0)),
        out_shape=jax.ShapeDtypeStruct((N, 40), f),
    )(u, m5, dinvb, We, be)

    return out


# speedup vs baseline: 1.1556x; 1.1556x over previous
import functools

import jax
import jax.numpy as jnp
from jax import lax
from jax.experimental import pallas as pl
from jax.experimental.pallas import tpu as pltpu
from jax.experimental.pallas import tpu_sc as plsc

N = 10000
F16 = 16
DEPTH = 12
LEAD = 6

NW = 32
CHUNK = 128
NCH = 80
EPAD = NW * NCH * CHUNK
ACC_N = 10240
SROWS = ACC_N // 16
BN = 1000

_mesh = plsc.VectorSubcoreMesh(core_axis_name="c", subcore_axis_name="s")
_sc_params = pltpu.CompilerParams(use_tc_tiling_on_sc=False)


PW = NCH * CHUNK
PE = 320000 // NW
PT = 320000 // 16


@functools.partial(
    pl.kernel,
    out_type=jax.ShapeDtypeStruct((4, ACC_N * 16), jnp.float32),
    mesh=_mesh,
    compiler_params=pltpu.CompilerParams(use_tc_tiling_on_sc=False,
                                         needs_layout_passes=False),
    scratch_types=[
        pltpu.VMEM_SHARED((16, ACC_N), jnp.float32),
        pltpu.VMEM((PT,), jnp.int32),
        pltpu.VMEM((ACC_N,), jnp.float32),
        pltpu.VMEM((16 * SROWS,), jnp.float32),
        pltpu.VMEM((SROWS * 16,), jnp.float32),
    ],
)
def _deg_kernel(d0, d1, d2, d3, out, part_sh, idx_v, cnt_v, tmp_v, stage_v):
    cid = lax.axis_index("c")
    sid = lax.axis_index("s")
    row0 = sid * SROWS

    z16 = jnp.zeros((16,), jnp.float32)
    one16 = jnp.ones((16,), jnp.float32)

    for k, dref in enumerate((d0, d1, d2, d3)):
        @pl.loop(0, ACC_N // 16)
        def _(i):
            cnt_v[pl.ds(i * 16, 16)] = z16

        pltpu.sync_copy(dref.at[1, pl.ds(sid * PT, PT)], idx_v)

        @pl.loop(0, PT // 64)
        def _(j):
            for l in range(4):
                idx16 = idx_v[pl.ds(j * 64 + l * 16, 16)]
                plsc.addupdate_scatter(cnt_v, [idx16], one16)

        for t16 in range((PT % 64) // 16):
            idx16 = idx_v[pl.ds((PT // 64) * 64 + t16 * 16, 16)]
            plsc.addupdate_scatter(cnt_v, [idx16], one16)

        pltpu.sync_copy(cnt_v, part_sh.at[sid])
        plsc.subcore_barrier()

        for t in range(16):
            pltpu.sync_copy(part_sh.at[t, pl.ds(row0, SROWS)],
                            tmp_v.at[pl.ds(t * SROWS, SROWS)])
        plsc.subcore_barrier()

        @pl.when(cid == (k // 2))
        def _():
            @pl.loop(0, SROWS // 16)
            def _(q):
                acc16 = tmp_v[pl.ds(q * 16, 16)]
                for t in range(1, 16):
                    acc16 = acc16 + tmp_v[pl.ds(t * SROWS + q * 16, 16)]
                d = 1.0 + acc16
                i32 = plsc.bitcast(d, jnp.int32)
                i32 = jnp.int32(0x5F3759DF) - (i32 >> 1)
                y = plsc.bitcast(i32, jnp.float32)
                for _ in range(3):
                    y = y * (1.5 - 0.5 * d * y * y)
                for e in range(16):
                    stage_v[pl.ds((q * 16 + e) * 16, 16)] = (
                        jnp.broadcast_to(y[e], (16,)))

            pltpu.sync_copy(stage_v,
                            out.at[k, pl.ds(row0 * 16, SROWS * 16)])


NCHF = PE // CHUNK
TAIL = PE - NCHF * CHUNK
_NITER = ((NCHF + LEAD + DEPTH - 1) // DEPTH) * DEPTH


def _make_msg_kernel(nsets, width):
    @functools.partial(
        pl.kernel,
        out_type=jax.ShapeDtypeStruct((2, nsets, ACC_N, width), jnp.float32),
        mesh=_mesh,
        compiler_params=_sc_params,
        scratch_types=[
            [pltpu.VMEM_SHARED((ACC_N, width), jnp.float32)
             for _ in range(nsets)],
            pltpu.VMEM((PE,), jnp.int32),
            pltpu.VMEM((PE,), jnp.int32),
            [pltpu.VMEM((CHUNK, width), jnp.float32) for _ in range(DEPTH)],
            pltpu.VMEM((TAIL, width), jnp.float32),
            pltpu.VMEM((SROWS, width), jnp.float32),
            pltpu.VMEM_SHARED((ACC_N, width), jnp.float32),
            [pltpu.SemaphoreType.DMA for _ in range(DEPTH)],
            [pltpu.SemaphoreType.DMA for _ in range(DEPTH)],
            pltpu.SemaphoreType.DMA,
        ],
    )
    def msg_kernel(*args):
        gs = args[:nsets]
        erefs = args[nsets:2 * nsets]
        out = args[2 * nsets]
        (accs, sidx_v, didx_v, bufs, tbuf, zeros_v, g_sh, gsems,
         ssems, tsem) = args[2 * nsets + 1:]

        cid = lax.axis_index("c")
        sid = lax.axis_index("s")
        wid = sid * 2 + cid
        row0 = sid * SROWS

        for w0 in range(0, width, 16):
            z = jnp.zeros((16,), jnp.float32)

            @pl.loop(0, SROWS)
            def _(i):
                zeros_v[i, pl.ds(w0, 16)] = z

        for acc in accs:
            pltpu.sync_copy(zeros_v, acc.at[pl.ds(row0, SROWS)])
        plsc.subcore_barrier()

        for g, eref, acc in zip(gs, erefs, accs):
            @pl.when(sid < 15)
            def _():
                pltpu.sync_copy(g.at[pl.ds(sid * 640, 640)],
                                g_sh.at[pl.ds(sid * 640, 640)])

            @pl.when(sid == 15)
            def _():
                pltpu.sync_copy(g.at[pl.ds(9600, N - 9600)],
                                g_sh.at[pl.ds(9600, N - 9600)])

            pltpu.sync_copy(eref.at[0, pl.ds(wid * PE, PE)], sidx_v)
            pltpu.sync_copy(eref.at[1, pl.ds(wid * PE, PE)], didx_v)
            plsc.subcore_barrier()

            def s_at(c):
                return sidx_v.at[pl.ds(pl.multiple_of(c * CHUNK, CHUNK),
                                       CHUNK)]

            def d_at(c):
                return didx_v.at[pl.ds(pl.multiple_of(c * CHUNK, CHUNK),
                                       CHUNK)]

            @pl.loop(0, _NITER, step=DEPTH)
            def _(j):
                for b in range(DEPTH):
                    jj = j + b

                    @pl.when(jj < NCHF)
                    def _():
                        @pl.when(jj >= DEPTH)
                        def _():
                            pltpu.make_async_copy(
                                bufs[b], acc.at[d_at(jj - DEPTH)],
                                ssems[b]).wait()

                        pltpu.async_copy(g_sh.at[s_at(jj)], bufs[b],
                                         gsems[b])

                    jk = jj - LEAD
                    b2 = (b - LEAD) % DEPTH

                    @pl.when((jk >= 0) & (jk < NCHF))
                    def _():
                        pltpu.make_async_copy(
                            g_sh.at[s_at(jk)], bufs[b2], gsems[b2]).wait()
                        pltpu.async_copy(bufs[b2], acc.at[d_at(jk)],
                                         ssems[b2], add=True)

            for b in range(DEPTH):
                c = NCHF - DEPTH + b
                pltpu.make_async_copy(
                    bufs[c % DEPTH], acc.at[d_at(c)], ssems[c % DEPTH]).wait()

            tidx_s = sidx_v.at[pl.ds(NCHF * CHUNK, TAIL)]
            tidx_d = didx_v.at[pl.ds(NCHF * CHUNK, TAIL)]
            pltpu.async_copy(g_sh.at[tidx_s], tbuf, tsem).wait()
            pltpu.async_copy(tbuf, acc.at[tidx_d], tsem, add=True).wait()

            plsc.subcore_barrier()
        for k, acc in enumerate(accs):
            pltpu.sync_copy(acc.at[pl.ds(row0, SROWS)],
                            out.at[cid, k, pl.ds(row0, SROWS)])

    return msg_kernel


_msg4 = _make_msg_kernel(4, F16)
_msg1 = _make_msg_kernel(1, F16)


def _tc1a_body(x0_ref, x1_ref, x2_ref, x3_ref, w1_ref, w2_ref, w3_ref,
               h0_ref, h1_ref, h2_ref, h3_ref):
    outs = (h0_ref, h1_ref, h2_ref, h3_ref)
    xs = (x0_ref, x1_ref, x2_ref, x3_ref)
    ws = (w1_ref, w2_ref, w3_ref, w2_ref)
    for k in range(4):
        outs[k][...] = jnp.dot(xs[k][...], ws[k][...],
                               preferred_element_type=jnp.float32)


def _tc1b_body(h0_ref, h1_ref, h2_ref, h3_ref, dinv_ref,
               g0_ref, g1_ref, g2_ref, g3_ref):
    outs = (g0_ref, g1_ref, g2_ref, g3_ref)
    hs = (h0_ref, h1_ref, h2_ref, h3_ref)
    for k in range(4):
        outs[k][...] = hs[k][...] * dinv_ref[k]


def _tc2_body(g0_ref, g1_ref, g2_ref, g3_ref, macc_ref, dinv_ref, bs_ref,
              u_ref):
    gr = (g0_ref, g1_ref, g2_ref, g3_ref)
    outs = []
    for k in range(4):
        acc = macc_ref[0, k] + macc_ref[1, k]
        outs.append(dinv_ref[k] * (acc + gr[k][...]) + bs_ref[k])
    xp = outs[0]
    x = jnp.maximum(xp, 0.0)
    xc1 = outs[1] + x + xp
    x = jnp.maximum(xc1, 0.0)
    xc2 = outs[2] + x + xc1
    x = jnp.maximum(xc2, 0.0)
    xf = outs[3] + x + xc1 + xc2
    x4 = jnp.maximum(xf, 0.0)
    u_ref[...] = x4 * dinv_ref[3]


def _tc3_body(u_ref, m5_ref, dinv_ref, we_ref, be_ref, o_ref):
    dinv = dinv_ref[0][:, 0:1]
    s = m5_ref[0, 0] + m5_ref[1, 0] + u_ref[...]
    h = jnp.dot(s, we_ref[...], preferred_element_type=jnp.float32)
    logits = dinv * h + be_ref[...]
    mx = jnp.max(logits, axis=1, keepdims=True)
    lse = mx + jnp.log(jnp.sum(jnp.exp(logits - mx), axis=1, keepdims=True))
    o_ref[...] = logits - lse


def kernel(x_parent, x_child1, x_child2, x_final_descendants,
           edge_index_parent, edge_index_child1, edge_index_child2,
           edge_index_final_descendants,
           W1, b1, W2, b2, W3, b3, We, be):
    f = jnp.float32
    bs = jnp.stack([b1, b2, b3, b2])

    dinvb = _deg_kernel(edge_index_parent, edge_index_child1,
                        edge_index_child2, edge_index_final_descendants)
    dinvb = dinvb.reshape(4, ACC_N, 16)

    F0, F1, F2 = W1.shape[0], W2.shape[0], W3.shape[0]
    h0, h1, h2, h3 = pl.pallas_call(
        _tc1a_body,
        grid=(N // BN,),
        in_specs=[
            pl.BlockSpec((BN, F0), lambda i: (i, 0)),
            pl.BlockSpec((BN, F1), lambda i: (i, 0)),
            pl.BlockSpec((BN, F2), lambda i: (i, 0)),
            pl.BlockSpec((BN, F1), lambda i: (i, 0)),
            pl.BlockSpec((F0, F16), lambda i: (0, 0)),
            pl.BlockSpec((F1, F16), lambda i: (0, 0)),
            pl.BlockSpec((F2, F16), lambda i: (0, 0)),
        ],
        out_specs=[pl.BlockSpec((BN, F16), lambda i: (i, 0))] * 4,
        out_shape=[jax.ShapeDtypeStruct((N, F16), f)] * 4,
    )(x_parent, x_child1, x_child2, x_final_descendants, W1, W2, W3)

    g0, g1, g2, g3 = pl.pallas_call(
        _tc1b_body,
        grid=(N // BN,),
        in_specs=[pl.BlockSpec((BN, F16), lambda i: (i, 0))] * 4 + [
            pl.BlockSpec((4, BN, F16), lambda i: (0, i, 0)),
        ],
        out_specs=[pl.BlockSpec((BN, F16), lambda i: (i, 0))] * 4,
        out_shape=[jax.ShapeDtypeStruct((N, F16), f)] * 4,
    )(h0, h1, h2, h3, dinvb)

    macc = _msg4(g0, g1, g2, g3, edge_index_parent, edge_index_child1,
                 edge_index_child2, edge_index_final_descendants)

    u = pl.pallas_call(
        _tc2_body,
        grid=(N // BN,),
        in_specs=[pl.BlockSpec((BN, F16), lambda i: (i, 0))] * 4 + [
            pl.BlockSpec((2, 4, BN, F16), lambda i: (0, 0, i, 0)),
            pl.BlockSpec((4, BN, F16), lambda i: (0, i, 0)),
            pl.BlockSpec((4, F16), lambda i: (0, 0)),
        ],
        out_specs=pl.BlockSpec((BN, F16), lambda i: (i, 0)),
        out_shape=jax.ShapeDtypeStruct((N, F16), f),
    )(g0, g1, g2, g3, macc, dinvb, bs)

    m5 = _msg1(u, edge_index_final_descendants)

    out = pl.pallas_call(
        _tc3_body,
        grid=(N // BN,),
        in_specs=[
            pl.BlockSpec((BN, F16), lambda i: (i, 0)),
            pl.BlockSpec((2, 1, BN, F16), lambda i: (0, 0, i, 0)),
            pl.BlockSpec((1, BN, F16), lambda i: (3, i, 0)),
            pl.BlockSpec((F16, 40), lambda i: (0, 0)),
            pl.BlockSpec((40,), lambda i: (0,)),
        ],
        out_specs=pl.BlockSpec((BN, 40), lambda i: (i, 0)),
        out_shape=jax.ShapeDtypeStruct((N, 40), f),
    )(u, m5, dinvb, We, be)

    return out


# speedup vs baseline: 1.1611x; 1.0048x over previous
import functools

import jax
import jax.numpy as jnp
from jax import lax
from jax.experimental import pallas as pl
from jax.experimental.pallas import tpu as pltpu
from jax.experimental.pallas import tpu_sc as plsc

N = 10000
F16 = 16
DEPTH = 12
LEAD = 6

NW = 32
CHUNK = 128
NCH = 80
EPAD = NW * NCH * CHUNK
ACC_N = 10240
SROWS = ACC_N // 16
BN = 2000

_mesh = plsc.VectorSubcoreMesh(core_axis_name="c", subcore_axis_name="s")
_sc_params = pltpu.CompilerParams(use_tc_tiling_on_sc=False)


PW = NCH * CHUNK
PE = 320000 // NW
PT = 320000 // 16


@functools.partial(
    pl.kernel,
    out_type=jax.ShapeDtypeStruct((4, ACC_N * 16), jnp.float32),
    mesh=_mesh,
    compiler_params=pltpu.CompilerParams(use_tc_tiling_on_sc=False,
                                         needs_layout_passes=False),
    scratch_types=[
        pltpu.VMEM_SHARED((16, ACC_N), jnp.float32),
        pltpu.VMEM((PT,), jnp.int32),
        pltpu.VMEM((ACC_N,), jnp.float32),
        pltpu.VMEM((16 * SROWS,), jnp.float32),
        pltpu.VMEM((SROWS * 16,), jnp.float32),
    ],
)
def _deg_kernel(d0, d1, d2, d3, out, part_sh, idx_v, cnt_v, tmp_v, stage_v):
    cid = lax.axis_index("c")
    sid = lax.axis_index("s")
    row0 = sid * SROWS

    z16 = jnp.zeros((16,), jnp.float32)
    one16 = jnp.ones((16,), jnp.float32)

    for k, dref in enumerate((d0, d1, d2, d3)):
        @pl.loop(0, ACC_N // 16)
        def _(i):
            cnt_v[pl.ds(i * 16, 16)] = z16

        pltpu.sync_copy(dref.at[1, pl.ds(sid * PT, PT)], idx_v)

        @pl.loop(0, PT // 64)
        def _(j):
            for l in range(4):
                idx16 = idx_v[pl.ds(j * 64 + l * 16, 16)]
                plsc.addupdate_scatter(cnt_v, [idx16], one16)

        for t16 in range((PT % 64) // 16):
            idx16 = idx_v[pl.ds((PT // 64) * 64 + t16 * 16, 16)]
            plsc.addupdate_scatter(cnt_v, [idx16], one16)

        pltpu.sync_copy(cnt_v, part_sh.at[sid])
        plsc.subcore_barrier()

        for t in range(16):
            pltpu.sync_copy(part_sh.at[t, pl.ds(row0, SROWS)],
                            tmp_v.at[pl.ds(t * SROWS, SROWS)])
        plsc.subcore_barrier()

        @pl.when(cid == (k // 2))
        def _():
            @pl.loop(0, SROWS // 16)
            def _(q):
                acc16 = tmp_v[pl.ds(q * 16, 16)]
                for t in range(1, 16):
                    acc16 = acc16 + tmp_v[pl.ds(t * SROWS + q * 16, 16)]
                d = 1.0 + acc16
                i32 = plsc.bitcast(d, jnp.int32)
                i32 = jnp.int32(0x5F3759DF) - (i32 >> 1)
                y = plsc.bitcast(i32, jnp.float32)
                for _ in range(3):
                    y = y * (1.5 - 0.5 * d * y * y)
                for e in range(16):
                    stage_v[pl.ds((q * 16 + e) * 16, 16)] = (
                        jnp.broadcast_to(y[e], (16,)))

            pltpu.sync_copy(stage_v,
                            out.at[k, pl.ds(row0 * 16, SROWS * 16)])


NCHF = PE // CHUNK
TAIL = PE - NCHF * CHUNK
_NITER = ((NCHF + LEAD + DEPTH - 1) // DEPTH) * DEPTH


def _make_msg_kernel(nsets, width):
    @functools.partial(
        pl.kernel,
        out_type=jax.ShapeDtypeStruct((2, nsets, ACC_N, width), jnp.float32),
        mesh=_mesh,
        compiler_params=_sc_params,
        scratch_types=[
            [pltpu.VMEM_SHARED((ACC_N, width), jnp.float32)
             for _ in range(nsets)],
            pltpu.VMEM((PE,), jnp.int32),
            pltpu.VMEM((PE,), jnp.int32),
            [pltpu.VMEM((CHUNK, width), jnp.float32) for _ in range(DEPTH)],
            pltpu.VMEM((TAIL, width), jnp.float32),
            pltpu.VMEM((SROWS, width), jnp.float32),
            pltpu.VMEM_SHARED((ACC_N, width), jnp.float32),
            [pltpu.SemaphoreType.DMA for _ in range(DEPTH)],
            [pltpu.SemaphoreType.DMA for _ in range(DEPTH)],
            pltpu.SemaphoreType.DMA,
        ],
    )
    def msg_kernel(*args):
        gs = args[:nsets]
        erefs = args[nsets:2 * nsets]
        out = args[2 * nsets]
        (accs, sidx_v, didx_v, bufs, tbuf, zeros_v, g_sh, gsems,
         ssems, tsem) = args[2 * nsets + 1:]

        cid = lax.axis_index("c")
        sid = lax.axis_index("s")
        wid = sid * 2 + cid
        row0 = sid * SROWS

        for w0 in range(0, width, 16):
            z = jnp.zeros((16,), jnp.float32)

            @pl.loop(0, SROWS)
            def _(i):
                zeros_v[i, pl.ds(w0, 16)] = z

        for acc in accs:
            pltpu.sync_copy(zeros_v, acc.at[pl.ds(row0, SROWS)])
        plsc.subcore_barrier()

        for g, eref, acc in zip(gs, erefs, accs):
            @pl.when(sid < 15)
            def _():
                pltpu.sync_copy(g.at[pl.ds(sid * 640, 640)],
                                g_sh.at[pl.ds(sid * 640, 640)])

            @pl.when(sid == 15)
            def _():
                pltpu.sync_copy(g.at[pl.ds(9600, N - 9600)],
                                g_sh.at[pl.ds(9600, N - 9600)])

            pltpu.sync_copy(eref.at[0, pl.ds(wid * PE, PE)], sidx_v)
            pltpu.sync_copy(eref.at[1, pl.ds(wid * PE, PE)], didx_v)
            plsc.subcore_barrier()

            def s_at(c):
                return sidx_v.at[pl.ds(pl.multiple_of(c * CHUNK, CHUNK),
                                       CHUNK)]

            def d_at(c):
                return didx_v.at[pl.ds(pl.multiple_of(c * CHUNK, CHUNK),
                                       CHUNK)]

            @pl.loop(0, _NITER, step=DEPTH)
            def _(j):
                for b in range(DEPTH):
                    jj = j + b

                    @pl.when(jj < NCHF)
                    def _():
                        @pl.when(jj >= DEPTH)
                        def _():
                            pltpu.make_async_copy(
                                bufs[b], acc.at[d_at(jj - DEPTH)],
                                ssems[b]).wait()

                        pltpu.async_copy(g_sh.at[s_at(jj)], bufs[b],
                                         gsems[b])

                    jk = jj - LEAD
                    b2 = (b - LEAD) % DEPTH

                    @pl.when((jk >= 0) & (jk < NCHF))
                    def _():
                        pltpu.make_async_copy(
                            g_sh.at[s_at(jk)], bufs[b2], gsems[b2]).wait()
                        pltpu.async_copy(bufs[b2], acc.at[d_at(jk)],
                                         ssems[b2], add=True)

            for b in range(DEPTH):
                c = NCHF - DEPTH + b
                pltpu.make_async_copy(
                    bufs[c % DEPTH], acc.at[d_at(c)], ssems[c % DEPTH]).wait()

            tidx_s = sidx_v.at[pl.ds(NCHF * CHUNK, TAIL)]
            tidx_d = didx_v.at[pl.ds(NCHF * CHUNK, TAIL)]
            pltpu.async_copy(g_sh.at[tidx_s], tbuf, tsem).wait()
            pltpu.async_copy(tbuf, acc.at[tidx_d], tsem, add=True).wait()

            plsc.subcore_barrier()
        for k, acc in enumerate(accs):
            pltpu.sync_copy(acc.at[pl.ds(row0, SROWS)],
                            out.at[cid, k, pl.ds(row0, SROWS)])

    return msg_kernel


_msg4 = _make_msg_kernel(4, F16)
_msg1 = _make_msg_kernel(1, F16)


def _tc1a_body(x0_ref, x1_ref, x2_ref, x3_ref, w1_ref, w2_ref, w3_ref,
               h0_ref, h1_ref, h2_ref, h3_ref):
    outs = (h0_ref, h1_ref, h2_ref, h3_ref)
    xs = (x0_ref, x1_ref, x2_ref, x3_ref)
    ws = (w1_ref, w2_ref, w3_ref, w2_ref)
    for k in range(4):
        outs[k][...] = jnp.dot(xs[k][...], ws[k][...],
                               preferred_element_type=jnp.float32)


def _tc1b_body(h0_ref, h1_ref, h2_ref, h3_ref, dinv_ref,
               g0_ref, g1_ref, g2_ref, g3_ref):
    outs = (g0_ref, g1_ref, g2_ref, g3_ref)
    hs = (h0_ref, h1_ref, h2_ref, h3_ref)
    for k in range(4):
        outs[k][...] = hs[k][...] * dinv_ref[k]


def _tc2_body(g0_ref, g1_ref, g2_ref, g3_ref, macc_ref, dinv_ref, bs_ref,
              u_ref):
    gr = (g0_ref, g1_ref, g2_ref, g3_ref)
    outs = []
    for k in range(4):
        acc = macc_ref[0, k] + macc_ref[1, k]
        outs.append(dinv_ref[k] * (acc + gr[k][...]) + bs_ref[k])
    xp = outs[0]
    x = jnp.maximum(xp, 0.0)
    xc1 = outs[1] + x + xp
    x = jnp.maximum(xc1, 0.0)
    xc2 = outs[2] + x + xc1
    x = jnp.maximum(xc2, 0.0)
    xf = outs[3] + x + xc1 + xc2
    x4 = jnp.maximum(xf, 0.0)
    u_ref[...] = x4 * dinv_ref[3]


def _tc3_body(u_ref, m5_ref, dinv_ref, we_ref, be_ref, o_ref):
    dinv = dinv_ref[0][:, 0:1]
    s = m5_ref[0, 0] + m5_ref[1, 0] + u_ref[...]
    h = jnp.dot(s, we_ref[...], preferred_element_type=jnp.float32)
    logits = dinv * h + be_ref[...]
    mx = jnp.max(logits, axis=1, keepdims=True)
    lse = mx + jnp.log(jnp.sum(jnp.exp(logits - mx), axis=1, keepdims=True))
    o_ref[...] = logits - lse


def kernel(x_parent, x_child1, x_child2, x_final_descendants,
           edge_index_parent, edge_index_child1, edge_index_child2,
           edge_index_final_descendants,
           W1, b1, W2, b2, W3, b3, We, be):
    f = jnp.float32
    bs = jnp.stack([b1, b2, b3, b2])

    dinvb = _deg_kernel(edge_index_parent, edge_index_child1,
                        edge_index_child2, edge_index_final_descendants)
    dinvb = dinvb.reshape(4, ACC_N, 16)

    F0, F1, F2 = W1.shape[0], W2.shape[0], W3.shape[0]
    h0, h1, h2, h3 = pl.pallas_call(
        _tc1a_body,
        grid=(N // BN,),
        in_specs=[
            pl.BlockSpec((BN, F0), lambda i: (i, 0)),
            pl.BlockSpec((BN, F1), lambda i: (i, 0)),
            pl.BlockSpec((BN, F2), lambda i: (i, 0)),
            pl.BlockSpec((BN, F1), lambda i: (i, 0)),
            pl.BlockSpec((F0, F16), lambda i: (0, 0)),
            pl.BlockSpec((F1, F16), lambda i: (0, 0)),
            pl.BlockSpec((F2, F16), lambda i: (0, 0)),
        ],
        out_specs=[pl.BlockSpec((BN, F16), lambda i: (i, 0))] * 4,
        out_shape=[jax.ShapeDtypeStruct((N, F16), f)] * 4,
    )(x_parent, x_child1, x_child2, x_final_descendants, W1, W2, W3)

    g0, g1, g2, g3 = pl.pallas_call(
        _tc1b_body,
        grid=(N // BN,),
        in_specs=[pl.BlockSpec((BN, F16), lambda i: (i, 0))] * 4 + [
            pl.BlockSpec((4, BN, F16), lambda i: (0, i, 0)),
        ],
        out_specs=[pl.BlockSpec((BN, F16), lambda i: (i, 0))] * 4,
        out_shape=[jax.ShapeDtypeStruct((N, F16), f)] * 4,
    )(h0, h1, h2, h3, dinvb)

    macc = _msg4(g0, g1, g2, g3, edge_index_parent, edge_index_child1,
                 edge_index_child2, edge_index_final_descendants)

    u = pl.pallas_call(
        _tc2_body,
        grid=(N // BN,),
        in_specs=[pl.BlockSpec((BN, F16), lambda i: (i, 0))] * 4 + [
            pl.BlockSpec((2, 4, BN, F16), lambda i: (0, 0, i, 0)),
            pl.BlockSpec((4, BN, F16), lambda i: (0, i, 0)),
            pl.BlockSpec((4, F16), lambda i: (0, 0)),
        ],
        out_specs=pl.BlockSpec((BN, F16), lambda i: (i, 0)),
        out_shape=jax.ShapeDtypeStruct((N, F16), f),
    )(g0, g1, g2, g3, macc, dinvb, bs)

    m5 = _msg1(u, edge_index_final_descendants)

    out = pl.pallas_call(
        _tc3_body,
        grid=(N // BN,),
        in_specs=[
            pl.BlockSpec((BN, F16), lambda i: (i, 0)),
            pl.BlockSpec((2, 1, BN, F16), lambda i: (0, 0, i, 0)),
            pl.BlockSpec((1, BN, F16), lambda i: (3, i, 0)),
            pl.BlockSpec((F16, 40), lambda i: (0, 0)),
            pl.BlockSpec((40,), lambda i: (0,)),
        ],
        out_specs=pl.BlockSpec((BN, 40), lambda i: (i, 0)),
        out_shape=jax.ShapeDtypeStruct((N, 40), f),
    )(u, m5, dinvb, We, be)

    return out
